# baseline (speedup 1.0000x reference)
import functools

import jax
import jax.numpy as jnp
from jax import lax
from jax.experimental import pallas as pl
from jax.experimental.pallas import tpu as pltpu
from jax.experimental.pallas import tpu_sc as plsc

NC = 2
NS = 16
NW = NC * NS
CHUNK = 64
BLK = 16
NBUF = 4


def _segsum_sc(h, src_p, dst_p, n_nodes, n_pad, ch):
    d = h.shape[1]
    rows_per_sub = n_pad // NS
    out_full = ((n_nodes + NS - 1) // NS + 7) // 8 * 8
    out_last = n_nodes - out_full * (NS - 1)
    assert 0 < out_last <= out_full and out_full % 8 == 0
    mesh = plsc.VectorSubcoreMesh(
        core_axis_name="c", subcore_axis_name="s", num_cores=NC, num_subcores=NS
    )

    @functools.partial(
        pl.kernel,
        out_type=jax.ShapeDtypeStruct((NC, n_nodes, d), jnp.float32),
        mesh=mesh,
        scratch_types=(
            [pltpu.VMEM((BLK, CHUNK), jnp.int32) for _ in range(3)]
            + [pltpu.VMEM((BLK, CHUNK), jnp.int32) for _ in range(3)]
            + [pltpu.VMEM((CHUNK, d), jnp.float32) for _ in range(NBUF)]
            + [pltpu.VMEM_SHARED((n_pad, d), jnp.float32)]
            + [pltpu.SemaphoreType.DMA] * (3 + 2 * NBUF)
        ),
    )
    def seg_kernel(h_hbm, src_hbm, dst_hbm, out_hbm,
                   sb0, sb1, sb2, db0, db1, db2, r0, r1, r2, r3, agg_sh,
                   si0, si1, si2, g0, g1, g2, g3, s0, s1, s2, s3):
        c = lax.axis_index("c")
        s = lax.axis_index("s")
        wid = c * NS + s
        nblk = ch // BLK
        idx_sets = ((sb0, db0, si0), (sb1, db1, si1), (sb2, db2, si2))
        rows = (r0, r1, r2, r3)
        gsem = (g0, g1, g2, g3)
        ssem = (s0, s1, s2, s3)

        def issue_idx(b):
            sb, db, smi = idx_sets[b % 3]
            c0 = pltpu.async_copy(src_hbm.at[wid, pl.ds(b * BLK, BLK)], sb, smi)
            c1 = pltpu.async_copy(dst_hbm.at[wid, pl.ds(b * BLK, BLK)], db, smi)
            return (c0, c1)

        pend = issue_idx(0)

        @pl.loop(0, CHUNK)
        def _zr(r):
            @pl.loop(0, d, step=16)
            def _zc(cc):
                r0[r, pl.ds(cc, 16)] = jnp.zeros((16,), jnp.float32)

        @pl.loop(0, rows_per_sub // CHUNK)
        def _zs(kz):
            pltpu.sync_copy(
                r0, agg_sh.at[pl.ds(s * rows_per_sub + kz * CHUNK, CHUNK)]
            )

        plsc.subcore_barrier()

        def _wait(buf, sem):
            pltpu.make_async_copy(h_hbm.at[pl.ds(0, CHUNK)], buf, sem).wait()

        for b in range(nblk):
            sb, db, _ = idx_sets[b % 3]
            dbp = idx_sets[(b - 1) % 3][1]
            pend[0].wait()
            pend[1].wait()
            if b + 1 < nblk:
                pend = issue_idx(b + 1)
            if b > 0:
                _wait(rows[2], gsem[2])
                pltpu.async_copy(rows[2], agg_sh.at[dbp.at[BLK - 2]], s2, add=True)
                _wait(rows[3], gsem[3])
                pltpu.async_copy(rows[3], agg_sh.at[dbp.at[BLK - 1]], s3, add=True)

            first = b == 0

            @pl.loop(0, BLK, step=NBUF)
            def _go(j, sb=sb, db=db, first=first):
                for k in range(NBUF):
                    k2 = (k + 2) % NBUF

                    def _free(k=k):
                        _wait(rows[k], ssem[k])

                    def _scat(k2=k2, k=k, db=db):
                        _wait(rows[k2], gsem[k2])
                        pltpu.async_copy(
                            rows[k2], agg_sh.at[db.at[j + k - 2]], ssem[k2],
                            add=True,
                        )

                    if first:
                        pl.when(j >= NBUF)(_free)
                    else:
                        _free()
                    if k < 2:
                        pl.when(j >= NBUF)(_scat)
                    pltpu.async_copy(h_hbm.at[sb.at[j + k]], rows[k], gsem[k])
                    if k >= 2:
                        _scat()

        dbl = idx_sets[(nblk - 1) % 3][1]
        _wait(rows[2], gsem[2])
        pltpu.async_copy(rows[2], agg_sh.at[dbl.at[BLK - 2]], s2, add=True)
        _wait(rows[3], gsem[3])
        pltpu.async_copy(rows[3], agg_sh.at[dbl.at[BLK - 1]], s3, add=True)
        for k in range(NBUF):
            _wait(rows[k], ssem[k])

        plsc.subcore_barrier()

        @pl.when(s < NS - 1)
        def _cp_full():
            pltpu.sync_copy(
                agg_sh.at[pl.ds(s * out_full, out_full)],
                out_hbm.at[c, pl.ds(s * out_full, out_full)],
            )

        @pl.when(s == NS - 1)
        def _cp_last():
            pltpu.sync_copy(
                agg_sh.at[pl.ds((NS - 1) * out_full, out_last)],
                out_hbm.at[c, pl.ds((NS - 1) * out_full, out_last)],
            )

    return seg_kernel(h, src_p, dst_p)


def _mlp_layer(x, p0, p1, W1, b1, W2, b2, block=1000):
    n, d = x.shape

    def body(x_r, p0_r, p1_r, w1_r, b1_r, w2_r, b2_r, o_r):
        z = x_r[...] + p0_r[...] + p1_r[...]
        h1 = jnp.maximum(
            jnp.dot(z, w1_r[...], preferred_element_type=jnp.float32) + b1_r[...], 0.0
        )
        h2 = jnp.dot(h1, w2_r[...], preferred_element_type=jnp.float32) + b2_r[...]
        o_r[...] = jnp.maximum(h2, 0.0)

    return pl.pallas_call(
        body,
        grid=(n // block,),
        in_specs=[
            pl.BlockSpec((block, d), lambda i: (i, 0)),
            pl.BlockSpec((block, d), lambda i: (i, 0)),
            pl.BlockSpec((block, d), lambda i: (i, 0)),
            pl.BlockSpec((d, d), lambda i: (0, 0)),
            pl.BlockSpec((1, d), lambda i: (0, 0)),
            pl.BlockSpec((d, d), lambda i: (0, 0)),
            pl.BlockSpec((1, d), lambda i: (0, 0)),
        ],
        out_specs=pl.BlockSpec((block, d), lambda i: (i, 0)),
        out_shape=jax.ShapeDtypeStruct((n, d), jnp.float32),
    )(x, p0, p1, W1, b1.reshape(1, d), W2, b2.reshape(1, d))


def _mlp_layer_head(x, p0, p1, W1, b1, W2, b2, Wh, bh, block=1000):
    n, d = x.shape

    def body(x_r, p0_r, p1_r, w1_r, b1_r, w2_r, b2_r, wh_r, bh_r, o_r):
        z = x_r[...] + p0_r[...] + p1_r[...]
        h1 = jnp.maximum(
            jnp.dot(z, w1_r[...], preferred_element_type=jnp.float32) + b1_r[...], 0.0
        )
        h2 = jnp.dot(h1, w2_r[...], preferred_element_type=jnp.float32) + b2_r[...]
        h2 = jnp.maximum(h2, 0.0)
        o_r[...] = jnp.dot(h2, wh_r[...], preferred_element_type=jnp.float32) + bh_r[...]

    return pl.pallas_call(
        body,
        grid=(n // block,),
        in_specs=[
            pl.BlockSpec((block, d), lambda i: (i, 0)),
            pl.BlockSpec((block, d), lambda i: (i, 0)),
            pl.BlockSpec((block, d), lambda i: (i, 0)),
            pl.BlockSpec((d, d), lambda i: (0, 0)),
            pl.BlockSpec((1, d), lambda i: (0, 0)),
            pl.BlockSpec((d, d), lambda i: (0, 0)),
            pl.BlockSpec((1, d), lambda i: (0, 0)),
            pl.BlockSpec((d, 1), lambda i: (0, 0)),
            pl.BlockSpec((1, 1), lambda i: (0, 0)),
        ],
        out_specs=pl.BlockSpec((block, 1), lambda i: (i, 0)),
        out_shape=jax.ShapeDtypeStruct((n, 1), jnp.float32),
    )(x, p0, p1, W1, b1.reshape(1, d), W2, b2.reshape(1, d), Wh, bh.reshape(1, 1))


def kernel(x, edge_index, W1_0, b1_0, W2_0, b2_0, W1_1, b1_1, W2_1, b2_1,
           W1_2, b1_2, W2_2, b2_2, Wh, bh):
    n, d = x.shape
    e = edge_index.shape[1]

    ch = (e + NW * CHUNK - 1) // (NW * CHUNK)
    ch = (ch + BLK - 1) // BLK * BLK
    n_pad = (n + 1 + NS * CHUNK - 1) // (NS * CHUNK) * (NS * CHUNK)

    src = edge_index[0].astype(jnp.int32)
    dst = edge_index[1].astype(jnp.int32)
    per_w = -(-e // NW)
    tail = NW * per_w - e
    src_w = jnp.concatenate([src, jnp.zeros((tail,), jnp.int32)]).reshape(NW, per_w)
    dst_w = jnp.concatenate([dst, jnp.full((tail,), n, jnp.int32)]).reshape(NW, per_w)
    padw = ch * CHUNK - per_w
    wids = jnp.arange(NW, dtype=jnp.int32)[:, None]
    lanes = jnp.arange(padw, dtype=jnp.int32)[None, :]
    pad_src = (wids * padw + lanes) % n
    pad_dst = n + (wids * 7 + lanes) % (n_pad - n)
    src_p = jnp.concatenate([src_w, pad_src], axis=1).reshape(NW, ch, CHUNK)
    dst_p = jnp.concatenate([dst_w, pad_dst], axis=1).reshape(NW, ch, CHUNK)

    h = x
    layers = [(W1_0, b1_0, W2_0, b2_0), (W1_1, b1_1, W2_1, b2_1)]
    for (W1, b1, W2, b2) in layers:
        p = _segsum_sc(h, src_p, dst_p, n, n_pad, ch)
        h = _mlp_layer(h, p[0], p[1], W1, b1, W2, b2)
    p = _segsum_sc(h, src_p, dst_p, n, n_pad, ch)
    out = _mlp_layer_head(h, p[0], p[1], W1_2, b1_2, W2_2, b2_2, Wh, bh)
    return (out.reshape(n), None)


# speedup vs baseline: 1.0768x; 1.0768x over previous
import functools

import jax
import jax.numpy as jnp
from jax import lax
from jax.experimental import pallas as pl
from jax.experimental.pallas import tpu as pltpu
from jax.experimental.pallas import tpu_sc as plsc

NC = 2
NS = 16
NW = NC * NS
CHUNK = 64
BLK = 16
NBUF = 4


def _segsum_sc(h, src_p, dst_p, n_nodes, n_pad, ch):
    d = h.shape[1]
    rows_per_sub = n_pad // NS
    out_full = ((n_nodes + NS - 1) // NS + 7) // 8 * 8
    out_last = n_nodes - out_full * (NS - 1)
    assert 0 < out_last <= out_full and out_full % 8 == 0
    mesh = plsc.VectorSubcoreMesh(
        core_axis_name="c", subcore_axis_name="s", num_cores=NC, num_subcores=NS
    )

    @functools.partial(
        pl.kernel,
        out_type=jax.ShapeDtypeStruct((NC, n_nodes, d), jnp.float32),
        mesh=mesh,
        scratch_types=(
            [pltpu.VMEM((BLK, CHUNK), jnp.int32) for _ in range(3)]
            + [pltpu.VMEM((BLK, CHUNK), jnp.int32) for _ in range(3)]
            + [pltpu.VMEM((CHUNK, d), jnp.float32) for _ in range(NBUF)]
            + [pltpu.VMEM_SHARED((n_pad, d), jnp.float32)]
            + [pltpu.SemaphoreType.DMA] * (3 + 2 * NBUF)
        ),
    )
    def seg_kernel(h_hbm, src_hbm, dst_hbm, out_hbm,
                   sb0, sb1, sb2, db0, db1, db2, r0, r1, r2, r3, agg_sh,
                   si0, si1, si2, g0, g1, g2, g3, s0, s1, s2, s3):
        c = lax.axis_index("c")
        s = lax.axis_index("s")
        wid = c * NS + s
        nblk = ch // BLK
        idx_sets = ((sb0, db0, si0), (sb1, db1, si1), (sb2, db2, si2))
        rows = (r0, r1, r2, r3)
        gsem = (g0, g1, g2, g3)
        ssem = (s0, s1, s2, s3)

        def issue_idx(b):
            sb, db, smi = idx_sets[b % 3]
            c0 = pltpu.async_copy(src_hbm.at[wid, pl.ds(b * BLK, BLK)], sb, smi)
            c1 = pltpu.async_copy(dst_hbm.at[wid, pl.ds(b * BLK, BLK)], db, smi)
            return (c0, c1)

        pend = issue_idx(0)

        @pl.loop(0, CHUNK)
        def _zr(r):
            @pl.loop(0, d, step=16)
            def _zc(cc):
                r0[r, pl.ds(cc, 16)] = jnp.zeros((16,), jnp.float32)

        @pl.loop(0, rows_per_sub // CHUNK)
        def _zs(kz):
            pltpu.sync_copy(
                r0, agg_sh.at[pl.ds(s * rows_per_sub + kz * CHUNK, CHUNK)]
            )

        plsc.subcore_barrier()

        def _wait(buf, sem):
            pltpu.make_async_copy(h_hbm.at[pl.ds(0, CHUNK)], buf, sem).wait()

        for b in range(nblk):
            sb, db, _ = idx_sets[b % 3]
            dbp = idx_sets[(b - 1) % 3][1]
            pend[0].wait()
            pend[1].wait()
            if b + 1 < nblk:
                pend = issue_idx(b + 1)
            if b > 0:
                _wait(rows[2], gsem[2])
                pltpu.async_copy(rows[2], agg_sh.at[dbp.at[BLK - 2]], s2, add=True)
                _wait(rows[3], gsem[3])
                pltpu.async_copy(rows[3], agg_sh.at[dbp.at[BLK - 1]], s3, add=True)

            first = b == 0

            @pl.loop(0, BLK, step=NBUF)
            def _go(j, sb=sb, db=db, first=first):
                for k in range(NBUF):
                    k2 = (k + 2) % NBUF

                    def _free(k=k):
                        _wait(rows[k], ssem[k])

                    def _scat(k2=k2, k=k, db=db):
                        _wait(rows[k2], gsem[k2])
                        pltpu.async_copy(
                            rows[k2], agg_sh.at[db.at[j + k - 2]], ssem[k2],
                            add=True,
                        )

                    if first:
                        pl.when(j >= NBUF)(_free)
                    else:
                        _free()
                    if k < 2:
                        pl.when(j >= NBUF)(_scat)
                    pltpu.async_copy(h_hbm.at[sb.at[j + k]], rows[k], gsem[k])
                    if k >= 2:
                        _scat()

        dbl = idx_sets[(nblk - 1) % 3][1]
        _wait(rows[2], gsem[2])
        pltpu.async_copy(rows[2], agg_sh.at[dbl.at[BLK - 2]], s2, add=True)
        _wait(rows[3], gsem[3])
        pltpu.async_copy(rows[3], agg_sh.at[dbl.at[BLK - 1]], s3, add=True)
        for k in range(NBUF):
            _wait(rows[k], ssem[k])

        plsc.subcore_barrier()

        @pl.when(s < NS - 1)
        def _cp_full():
            pltpu.sync_copy(
                agg_sh.at[pl.ds(s * out_full, out_full)],
                out_hbm.at[c, pl.ds(s * out_full, out_full)],
            )

        @pl.when(s == NS - 1)
        def _cp_last():
            pltpu.sync_copy(
                agg_sh.at[pl.ds((NS - 1) * out_full, out_last)],
                out_hbm.at[c, pl.ds((NS - 1) * out_full, out_last)],
            )

    return seg_kernel(h, src_p, dst_p)


def _mlp_layer(x, p, W1, b1, W2, b2, block=2000):
    n, d = x.shape

    def body(x_r, p_r, w1_r, b1_r, w2_r, b2_r, o_r):
        z = x_r[...] + p_r[0] + p_r[1]
        h1 = jnp.maximum(
            jnp.dot(z, w1_r[...], preferred_element_type=jnp.float32) + b1_r[...], 0.0
        )
        h2 = jnp.dot(h1, w2_r[...], preferred_element_type=jnp.float32) + b2_r[...]
        o_r[...] = jnp.maximum(h2, 0.0)

    return pl.pallas_call(
        body,
        grid=(n // block,),
        in_specs=[
            pl.BlockSpec((block, d), lambda i: (i, 0)),
            pl.BlockSpec((NC, block, d), lambda i: (0, i, 0)),
            pl.BlockSpec((d, d), lambda i: (0, 0)),
            pl.BlockSpec((1, d), lambda i: (0, 0)),
            pl.BlockSpec((d, d), lambda i: (0, 0)),
            pl.BlockSpec((1, d), lambda i: (0, 0)),
        ],
        out_specs=pl.BlockSpec((block, d), lambda i: (i, 0)),
        out_shape=jax.ShapeDtypeStruct((n, d), jnp.float32),
    )(x, p, W1, b1.reshape(1, d), W2, b2.reshape(1, d))


def _mlp_layer_head(x, p, W1, b1, W2, b2, Wh, bh, block=2000):
    n, d = x.shape

    def body(x_r, p_r, w1_r, b1_r, w2_r, b2_r, wh_r, bh_r, o_r):
        z = x_r[...] + p_r[0] + p_r[1]
        h1 = jnp.maximum(
            jnp.dot(z, w1_r[...], preferred_element_type=jnp.float32) + b1_r[...], 0.0
        )
        h2 = jnp.dot(h1, w2_r[...], preferred_element_type=jnp.float32) + b2_r[...]
        h2 = jnp.maximum(h2, 0.0)
        o_r[...] = jnp.dot(h2, wh_r[...], preferred_element_type=jnp.float32) + bh_r[...]

    return pl.pallas_call(
        body,
        grid=(n // block,),
        in_specs=[
            pl.BlockSpec((block, d), lambda i: (i, 0)),
            pl.BlockSpec((NC, block, d), lambda i: (0, i, 0)),
            pl.BlockSpec((d, d), lambda i: (0, 0)),
            pl.BlockSpec((1, d), lambda i: (0, 0)),
            pl.BlockSpec((d, d), lambda i: (0, 0)),
            pl.BlockSpec((1, d), lambda i: (0, 0)),
            pl.BlockSpec((d, 1), lambda i: (0, 0)),
            pl.BlockSpec((1, 1), lambda i: (0, 0)),
        ],
        out_specs=pl.BlockSpec((block, 1), lambda i: (i, 0)),
        out_shape=jax.ShapeDtypeStruct((n, 1), jnp.float32),
    )(x, p, W1, b1.reshape(1, d), W2, b2.reshape(1, d), Wh, bh.reshape(1, 1))


def kernel(x, edge_index, W1_0, b1_0, W2_0, b2_0, W1_1, b1_1, W2_1, b2_1,
           W1_2, b1_2, W2_2, b2_2, Wh, bh):
    n, d = x.shape
    e = edge_index.shape[1]

    ch = (e + NW * CHUNK - 1) // (NW * CHUNK)
    ch = (ch + BLK - 1) // BLK * BLK
    n_pad = (n + 1 + NS * CHUNK - 1) // (NS * CHUNK) * (NS * CHUNK)

    src = edge_index[0].astype(jnp.int32)
    dst = edge_index[1].astype(jnp.int32)
    per_w = -(-e // NW)
    tail = NW * per_w - e
    src_w = jnp.concatenate([src, jnp.zeros((tail,), jnp.int32)]).reshape(NW, per_w)
    dst_w = jnp.concatenate([dst, jnp.full((tail,), n, jnp.int32)]).reshape(NW, per_w)
    padw = ch * CHUNK - per_w
    wids = jnp.arange(NW, dtype=jnp.int32)[:, None]
    lanes = jnp.arange(padw, dtype=jnp.int32)[None, :]
    pad_src = (wids * padw + lanes) % n
    pad_dst = n + (wids * 7 + lanes) % (n_pad - n)
    src_p = jnp.concatenate([src_w, pad_src], axis=1).reshape(NW, ch, CHUNK)
    dst_p = jnp.concatenate([dst_w, pad_dst], axis=1).reshape(NW, ch, CHUNK)

    h = x
    layers = [(W1_0, b1_0, W2_0, b2_0), (W1_1, b1_1, W2_1, b2_1)]
    for (W1, b1, W2, b2) in layers:
        p = _segsum_sc(h, src_p, dst_p, n, n_pad, ch)
        h = _mlp_layer(h, p, W1, b1, W2, b2)
    p = _segsum_sc(h, src_p, dst_p, n, n_pad, ch)
    out = _mlp_layer_head(h, p, W1_2, b1_2, W2_2, b2_2, Wh, bh)
    return (out.reshape(n), None)


# speedup vs baseline: 1.0938x; 1.0158x over previous
import functools

import jax
import jax.numpy as jnp
from jax import lax
from jax.experimental import pallas as pl
from jax.experimental.pallas import tpu as pltpu
from jax.experimental.pallas import tpu_sc as plsc

NC = 2
NS = 16
NW = NC * NS
CHUNK = 64
BLK = 16
NBUF = 4


def _segsum_sc(h, src_p, dst_p, psrc, pdst, n_nodes, n_pad, ch):
    d = h.shape[1]
    per_w = src_p.shape[0] // NW
    assert per_w * NW == src_p.shape[0] and per_w % 8 == 0
    assert (per_w - (ch // BLK - 1) * BLK * CHUNK) % 8 == 0
    rows_per_sub = n_pad // NS
    out_full = ((n_nodes + NS - 1) // NS + 7) // 8 * 8
    out_last = n_nodes - out_full * (NS - 1)
    assert 0 < out_last <= out_full and out_full % 8 == 0
    mesh = plsc.VectorSubcoreMesh(
        core_axis_name="c", subcore_axis_name="s", num_cores=NC, num_subcores=NS
    )

    @functools.partial(
        pl.kernel,
        out_type=jax.ShapeDtypeStruct((NC, n_nodes, d), jnp.float32),
        mesh=mesh,
        scratch_types=(
            [pltpu.VMEM((BLK * CHUNK,), jnp.int32) for _ in range(3)]
            + [pltpu.VMEM((BLK * CHUNK,), jnp.int32) for _ in range(3)]
            + [pltpu.VMEM((CHUNK, d), jnp.float32) for _ in range(NBUF)]
            + [pltpu.VMEM_SHARED((n_pad, d), jnp.float32)]
            + [pltpu.SemaphoreType.DMA] * (3 + 2 * NBUF)
        ),
    )
    def seg_kernel(h_hbm, src_hbm, dst_hbm, psrc_hbm, pdst_hbm, out_hbm,
                   sb0, sb1, sb2, db0, db1, db2, r0, r1, r2, r3, agg_sh,
                   si0, si1, si2, g0, g1, g2, g3, s0, s1, s2, s3):
        c = lax.axis_index("c")
        s = lax.axis_index("s")
        wid = c * NS + s
        nblk = ch // BLK
        padw = ch * CHUNK - per_w
        tail_real = per_w - (nblk - 1) * BLK * CHUNK
        idx_sets = ((sb0, db0, si0), (sb1, db1, si1), (sb2, db2, si2))
        rows = (r0, r1, r2, r3)
        gsem = (g0, g1, g2, g3)
        ssem = (s0, s1, s2, s3)

        def issue_idx(b):
            sb, db, smi = idx_sets[b % 3]
            base = wid * per_w + b * BLK * CHUNK
            if b < nblk - 1:
                c0 = pltpu.async_copy(src_hbm.at[pl.ds(base, BLK * CHUNK)], sb, smi)
                c1 = pltpu.async_copy(dst_hbm.at[pl.ds(base, BLK * CHUNK)], db, smi)
                return (c0, c1)
            c0 = pltpu.async_copy(
                src_hbm.at[pl.ds(base, tail_real)], sb.at[pl.ds(0, tail_real)], smi)
            c1 = pltpu.async_copy(
                dst_hbm.at[pl.ds(base, tail_real)], db.at[pl.ds(0, tail_real)], smi)
            c2 = pltpu.async_copy(
                psrc_hbm.at[pl.ds(wid * padw, padw)], sb.at[pl.ds(tail_real, padw)], smi)
            c3 = pltpu.async_copy(
                pdst_hbm.at[pl.ds(wid * padw, padw)], db.at[pl.ds(tail_real, padw)], smi)
            return (c0, c1, c2, c3)

        pend = issue_idx(0)

        @pl.loop(0, CHUNK)
        def _zr(r):
            @pl.loop(0, d, step=16)
            def _zc(cc):
                r0[r, pl.ds(cc, 16)] = jnp.zeros((16,), jnp.float32)

        @pl.loop(0, rows_per_sub // CHUNK)
        def _zs(kz):
            pltpu.sync_copy(
                r0, agg_sh.at[pl.ds(s * rows_per_sub + kz * CHUNK, CHUNK)]
            )

        plsc.subcore_barrier()

        def _wait(buf, sem):
            pltpu.make_async_copy(h_hbm.at[pl.ds(0, CHUNK)], buf, sem).wait()

        def dslice(buf, i):
            return buf.at[pl.ds(i * CHUNK, CHUNK)]

        for b in range(nblk):
            sb, db, _ = idx_sets[b % 3]
            dbp = idx_sets[(b - 1) % 3][1]
            for p in pend:
                p.wait()
            if b + 1 < nblk:
                pend = issue_idx(b + 1)
            if b > 0:
                _wait(rows[2], gsem[2])
                pltpu.async_copy(rows[2], agg_sh.at[dslice(dbp, BLK - 2)], s2, add=True)
                _wait(rows[3], gsem[3])
                pltpu.async_copy(rows[3], agg_sh.at[dslice(dbp, BLK - 1)], s3, add=True)

            first = b == 0

            @pl.loop(0, BLK, step=NBUF)
            def _go(j, sb=sb, db=db, first=first):
                for k in range(NBUF):
                    k2 = (k + 2) % NBUF

                    def _free(k=k):
                        _wait(rows[k], ssem[k])

                    def _scat(k2=k2, k=k, db=db):
                        _wait(rows[k2], gsem[k2])
                        pltpu.async_copy(
                            rows[k2], agg_sh.at[dslice(db, j + k - 2)], ssem[k2],
                            add=True,
                        )

                    if first:
                        pl.when(j >= NBUF)(_free)
                    else:
                        _free()
                    if k < 2:
                        pl.when(j >= NBUF)(_scat)
                    pltpu.async_copy(h_hbm.at[dslice(sb, j + k)], rows[k], gsem[k])
                    if k >= 2:
                        _scat()

        dbl = idx_sets[(nblk - 1) % 3][1]
        _wait(rows[2], gsem[2])
        pltpu.async_copy(rows[2], agg_sh.at[dslice(dbl, BLK - 2)], s2, add=True)
        _wait(rows[3], gsem[3])
        pltpu.async_copy(rows[3], agg_sh.at[dslice(dbl, BLK - 1)], s3, add=True)
        for k in range(NBUF):
            _wait(rows[k], ssem[k])

        plsc.subcore_barrier()

        @pl.when(s < NS - 1)
        def _cp_full():
            pltpu.sync_copy(
                agg_sh.at[pl.ds(s * out_full, out_full)],
                out_hbm.at[c, pl.ds(s * out_full, out_full)],
            )

        @pl.when(s == NS - 1)
        def _cp_last():
            pltpu.sync_copy(
                agg_sh.at[pl.ds((NS - 1) * out_full, out_last)],
                out_hbm.at[c, pl.ds((NS - 1) * out_full, out_last)],
            )

    return seg_kernel(h, src_p, dst_p, psrc, pdst)


def _mlp_layer(x, p, W1, b1, W2, b2, block=2000):
    n, d = x.shape

    def body(x_r, p_r, w1_r, b1_r, w2_r, b2_r, o_r):
        z = x_r[...] + p_r[0] + p_r[1]
        h1 = jnp.maximum(
            jnp.dot(z, w1_r[...], preferred_element_type=jnp.float32) + b1_r[...], 0.0
        )
        h2 = jnp.dot(h1, w2_r[...], preferred_element_type=jnp.float32) + b2_r[...]
        o_r[...] = jnp.maximum(h2, 0.0)

    return pl.pallas_call(
        body,
        grid=(n // block,),
        in_specs=[
            pl.BlockSpec((block, d), lambda i: (i, 0)),
            pl.BlockSpec((NC, block, d), lambda i: (0, i, 0)),
            pl.BlockSpec((d, d), lambda i: (0, 0)),
            pl.BlockSpec((1, d), lambda i: (0, 0)),
            pl.BlockSpec((d, d), lambda i: (0, 0)),
            pl.BlockSpec((1, d), lambda i: (0, 0)),
        ],
        out_specs=pl.BlockSpec((block, d), lambda i: (i, 0)),
        out_shape=jax.ShapeDtypeStruct((n, d), jnp.float32),
    )(x, p, W1, b1.reshape(1, d), W2, b2.reshape(1, d))


def _mlp_layer_head(x, p, W1, b1, W2, b2, Wh, bh, block=2000):
    n, d = x.shape

    def body(x_r, p_r, w1_r, b1_r, w2_r, b2_r, wh_r, bh_r, o_r):
        z = x_r[...] + p_r[0] + p_r[1]
        h1 = jnp.maximum(
            jnp.dot(z, w1_r[...], preferred_element_type=jnp.float32) + b1_r[...], 0.0
        )
        h2 = jnp.dot(h1, w2_r[...], preferred_element_type=jnp.float32) + b2_r[...]
        h2 = jnp.maximum(h2, 0.0)
        o_r[...] = jnp.dot(h2, wh_r[...], preferred_element_type=jnp.float32) + bh_r[...]

    return pl.pallas_call(
        body,
        grid=(n // block,),
        in_specs=[
            pl.BlockSpec((block, d), lambda i: (i, 0)),
            pl.BlockSpec((NC, block, d), lambda i: (0, i, 0)),
            pl.BlockSpec((d, d), lambda i: (0, 0)),
            pl.BlockSpec((1, d), lambda i: (0, 0)),
            pl.BlockSpec((d, d), lambda i: (0, 0)),
            pl.BlockSpec((1, d), lambda i: (0, 0)),
            pl.BlockSpec((d, 1), lambda i: (0, 0)),
            pl.BlockSpec((1, 1), lambda i: (0, 0)),
        ],
        out_specs=pl.BlockSpec((block, 1), lambda i: (i, 0)),
        out_shape=jax.ShapeDtypeStruct((n, 1), jnp.float32),
    )(x, p, W1, b1.reshape(1, d), W2, b2.reshape(1, d), Wh, bh.reshape(1, 1))


def kernel(x, edge_index, W1_0, b1_0, W2_0, b2_0, W1_1, b1_1, W2_1, b2_1,
           W1_2, b1_2, W2_2, b2_2, Wh, bh):
    n, d = x.shape
    e = edge_index.shape[1]

    ch = (e + NW * CHUNK - 1) // (NW * CHUNK)
    ch = (ch + BLK - 1) // BLK * BLK
    n_pad = (n + 1 + NS * CHUNK - 1) // (NS * CHUNK) * (NS * CHUNK)

    src = edge_index[0].reshape(e).astype(jnp.int32)
    dst = edge_index[1].reshape(e).astype(jnp.int32)
    per_w = e // NW
    padw = ch * CHUNK - per_w
    wids = jnp.arange(NW, dtype=jnp.int32)[:, None]
    lanes = jnp.arange(padw, dtype=jnp.int32)[None, :]
    pad_src = ((wids * padw + lanes) % n).reshape(NW * padw)
    pad_dst = (n + (wids * 7 + lanes) % (n_pad - n)).reshape(NW * padw)

    h = x
    layers = [(W1_0, b1_0, W2_0, b2_0), (W1_1, b1_1, W2_1, b2_1)]
    for (W1, b1, W2, b2) in layers:
        p = _segsum_sc(h, src, dst, pad_src, pad_dst, n, n_pad, ch)
        h = _mlp_layer(h, p, W1, b1, W2, b2)
    p = _segsum_sc(h, src, dst, pad_src, pad_dst, n, n_pad, ch)
    out = _mlp_layer_head(h, p, W1_2, b1_2, W2_2, b2_2, Wh, bh)
    return (out.reshape(n), None)


# speedup vs baseline: 1.0964x; 1.0024x over previous
import functools

import jax
import jax.numpy as jnp
from jax import lax
from jax.experimental import pallas as pl
from jax.experimental.pallas import tpu as pltpu
from jax.experimental.pallas import tpu_sc as plsc

NC = 2
NS = 16
NW = NC * NS
CHUNK = 64
BLK = 16
NBUF = 4


def _segsum_sc(h, src_p, dst_p, psrc, pdst, n_nodes, n_pad, ch):
    d = h.shape[1]
    per_w = src_p.shape[0] // NW
    assert per_w * NW == src_p.shape[0] and per_w % 8 == 0
    assert (per_w - (ch // BLK - 1) * BLK * CHUNK) % 8 == 0
    rows_per_sub = n_pad // NS
    out_full = ((n_nodes + NS - 1) // NS + 7) // 8 * 8
    out_last = n_nodes - out_full * (NS - 1)
    assert 0 < out_last <= out_full and out_full % 8 == 0
    mesh = plsc.VectorSubcoreMesh(
        core_axis_name="c", subcore_axis_name="s", num_cores=NC, num_subcores=NS
    )

    @functools.partial(
        pl.kernel,
        out_type=jax.ShapeDtypeStruct((NC, n_nodes, d), jnp.float32),
        mesh=mesh,
        scratch_types=(
            [pltpu.VMEM((BLK * CHUNK,), jnp.int32) for _ in range(3)]
            + [pltpu.VMEM((BLK * CHUNK,), jnp.int32) for _ in range(3)]
            + [pltpu.VMEM((CHUNK, d), jnp.float32) for _ in range(NBUF)]
            + [pltpu.VMEM_SHARED((n_pad, d), jnp.float32)]
            + [pltpu.SemaphoreType.DMA] * (4 + 2 * NBUF)
        ),
    )
    def seg_kernel(h_hbm, src_hbm, dst_hbm, psrc_hbm, pdst_hbm, out_hbm,
                   sb0, sb1, sb2, db0, db1, db2, r0, r1, r2, r3, agg_sh,
                   si0, si1, si2, zsem, g0, g1, g2, g3, s0, s1, s2, s3):
        c = lax.axis_index("c")
        s = lax.axis_index("s")
        wid = c * NS + s
        nblk = ch // BLK
        padw = ch * CHUNK - per_w
        tail_real = per_w - (nblk - 1) * BLK * CHUNK
        idx_sets = ((sb0, db0, si0), (sb1, db1, si1), (sb2, db2, si2))
        rows = (r0, r1, r2, r3)
        gsem = (g0, g1, g2, g3)
        ssem = (s0, s1, s2, s3)

        def issue_idx(b):
            sb, db, smi = idx_sets[b % 3]
            base = wid * per_w + b * BLK * CHUNK
            if b < nblk - 1:
                c0 = pltpu.async_copy(src_hbm.at[pl.ds(base, BLK * CHUNK)], sb, smi)
                c1 = pltpu.async_copy(dst_hbm.at[pl.ds(base, BLK * CHUNK)], db, smi)
                return (c0, c1)
            c0 = pltpu.async_copy(
                src_hbm.at[pl.ds(base, tail_real)], sb.at[pl.ds(0, tail_real)], smi)
            c1 = pltpu.async_copy(
                dst_hbm.at[pl.ds(base, tail_real)], db.at[pl.ds(0, tail_real)], smi)
            c2 = pltpu.async_copy(
                psrc_hbm.at[pl.ds(wid * padw, padw)], sb.at[pl.ds(tail_real, padw)], smi)
            c3 = pltpu.async_copy(
                pdst_hbm.at[pl.ds(wid * padw, padw)], db.at[pl.ds(tail_real, padw)], smi)
            return (c0, c1, c2, c3)

        pend = issue_idx(0)

        @pl.loop(0, CHUNK)
        def _zr(r):
            @pl.loop(0, d, step=16)
            def _zc(cc):
                r0[r, pl.ds(cc, 16)] = jnp.zeros((16,), jnp.float32)

        @pl.loop(0, rows_per_sub // CHUNK)
        def _zs(kz):
            pltpu.async_copy(
                r0, agg_sh.at[pl.ds(s * rows_per_sub + kz * CHUNK, CHUNK)], zsem
            )

        @pl.loop(0, rows_per_sub // CHUNK)
        def _zw(kz):
            pltpu.make_async_copy(
                r0, agg_sh.at[pl.ds(s * rows_per_sub, CHUNK)], zsem
            ).wait()

        plsc.subcore_barrier()

        def _wait(buf, sem):
            pltpu.make_async_copy(h_hbm.at[pl.ds(0, CHUNK)], buf, sem).wait()

        def dslice(buf, i):
            return buf.at[pl.ds(i * CHUNK, CHUNK)]

        for b in range(nblk):
            sb, db, _ = idx_sets[b % 3]
            dbp = idx_sets[(b - 1) % 3][1]
            for p in pend:
                p.wait()
            if b + 1 < nblk:
                pend = issue_idx(b + 1)
            if b > 0:
                _wait(rows[2], gsem[2])
                pltpu.async_copy(rows[2], agg_sh.at[dslice(dbp, BLK - 2)], s2, add=True)
                _wait(rows[3], gsem[3])
                pltpu.async_copy(rows[3], agg_sh.at[dslice(dbp, BLK - 1)], s3, add=True)

            first = b == 0

            @pl.loop(0, BLK, step=NBUF)
            def _go(j, sb=sb, db=db, first=first):
                for k in range(NBUF):
                    k2 = (k + 2) % NBUF

                    def _free(k=k):
                        _wait(rows[k], ssem[k])

                    def _scat(k2=k2, k=k, db=db):
                        _wait(rows[k2], gsem[k2])
                        pltpu.async_copy(
                            rows[k2], agg_sh.at[dslice(db, j + k - 2)], ssem[k2],
                            add=True,
                        )

                    if first:
                        pl.when(j >= NBUF)(_free)
                    else:
                        _free()
                    if k < 2:
                        pl.when(j >= NBUF)(_scat)
                    pltpu.async_copy(h_hbm.at[dslice(sb, j + k)], rows[k], gsem[k])
                    if k >= 2:
                        _scat()

        dbl = idx_sets[(nblk - 1) % 3][1]
        _wait(rows[2], gsem[2])
        pltpu.async_copy(rows[2], agg_sh.at[dslice(dbl, BLK - 2)], s2, add=True)
        _wait(rows[3], gsem[3])
        pltpu.async_copy(rows[3], agg_sh.at[dslice(dbl, BLK - 1)], s3, add=True)
        for k in range(NBUF):
            _wait(rows[k], ssem[k])

        plsc.subcore_barrier()

        @pl.when(s < NS - 1)
        def _cp_full():
            pltpu.sync_copy(
                agg_sh.at[pl.ds(s * out_full, out_full)],
                out_hbm.at[c, pl.ds(s * out_full, out_full)],
            )

        @pl.when(s == NS - 1)
        def _cp_last():
            pltpu.sync_copy(
                agg_sh.at[pl.ds((NS - 1) * out_full, out_last)],
                out_hbm.at[c, pl.ds((NS - 1) * out_full, out_last)],
            )

    return seg_kernel(h, src_p, dst_p, psrc, pdst)


def _mlp_layer(x, p, W1, b1, W2, b2, block=2000):
    n, d = x.shape

    def body(x_r, p_r, w1_r, b1_r, w2_r, b2_r, o_r):
        z = x_r[...] + p_r[0] + p_r[1]
        h1 = jnp.maximum(
            jnp.dot(z, w1_r[...], preferred_element_type=jnp.float32) + b1_r[...], 0.0
        )
        h2 = jnp.dot(h1, w2_r[...], preferred_element_type=jnp.float32) + b2_r[...]
        o_r[...] = jnp.maximum(h2, 0.0)

    return pl.pallas_call(
        body,
        grid=(n // block,),
        in_specs=[
            pl.BlockSpec((block, d), lambda i: (i, 0)),
            pl.BlockSpec((NC, block, d), lambda i: (0, i, 0)),
            pl.BlockSpec((d, d), lambda i: (0, 0)),
            pl.BlockSpec((1, d), lambda i: (0, 0)),
            pl.BlockSpec((d, d), lambda i: (0, 0)),
            pl.BlockSpec((1, d), lambda i: (0, 0)),
        ],
        out_specs=pl.BlockSpec((block, d), lambda i: (i, 0)),
        out_shape=jax.ShapeDtypeStruct((n, d), jnp.float32),
    )(x, p, W1, b1.reshape(1, d), W2, b2.reshape(1, d))


def _mlp_layer_head(x, p, W1, b1, W2, b2, Wh, bh, block=2000):
    n, d = x.shape

    def body(x_r, p_r, w1_r, b1_r, w2_r, b2_r, wh_r, bh_r, o_r):
        z = x_r[...] + p_r[0] + p_r[1]
        h1 = jnp.maximum(
            jnp.dot(z, w1_r[...], preferred_element_type=jnp.float32) + b1_r[...], 0.0
        )
        h2 = jnp.dot(h1, w2_r[...], preferred_element_type=jnp.float32) + b2_r[...]
        h2 = jnp.maximum(h2, 0.0)
        o_r[...] = jnp.dot(h2, wh_r[...], preferred_element_type=jnp.float32) + bh_r[...]

    return pl.pallas_call(
        body,
        grid=(n // block,),
        in_specs=[
            pl.BlockSpec((block, d), lambda i: (i, 0)),
            pl.BlockSpec((NC, block, d), lambda i: (0, i, 0)),
            pl.BlockSpec((d, d), lambda i: (0, 0)),
            pl.BlockSpec((1, d), lambda i: (0, 0)),
            pl.BlockSpec((d, d), lambda i: (0, 0)),
            pl.BlockSpec((1, d), lambda i: (0, 0)),
            pl.BlockSpec((d, 1), lambda i: (0, 0)),
            pl.BlockSpec((1, 1), lambda i: (0, 0)),
        ],
        out_specs=pl.BlockSpec((block, 1), lambda i: (i, 0)),
        out_shape=jax.ShapeDtypeStruct((n, 1), jnp.float32),
    )(x, p, W1, b1.reshape(1, d), W2, b2.reshape(1, d), Wh, bh.reshape(1, 1))


def kernel(x, edge_index, W1_0, b1_0, W2_0, b2_0, W1_1, b1_1, W2_1, b2_1,
           W1_2, b1_2, W2_2, b2_2, Wh, bh):
    n, d = x.shape
    e = edge_index.shape[1]

    ch = (e + NW * CHUNK - 1) // (NW * CHUNK)
    ch = (ch + BLK - 1) // BLK * BLK
    n_pad = (n + 1 + NS * CHUNK - 1) // (NS * CHUNK) * (NS * CHUNK)

    src = edge_index[0].reshape(e).astype(jnp.int32)
    dst = edge_index[1].reshape(e).astype(jnp.int32)
    per_w = e // NW
    padw = ch * CHUNK - per_w
    wids = jnp.arange(NW, dtype=jnp.int32)[:, None]
    lanes = jnp.arange(padw, dtype=jnp.int32)[None, :]
    pad_src = ((wids * padw + lanes) % n).reshape(NW * padw)
    pad_dst = (n + (wids * 7 + lanes) % (n_pad - n)).reshape(NW * padw)

    h = x
    layers = [(W1_0, b1_0, W2_0, b2_0), (W1_1, b1_1, W2_1, b2_1)]
    for (W1, b1, W2, b2) in layers:
        p = _segsum_sc(h, src, dst, pad_src, pad_dst, n, n_pad, ch)
        h = _mlp_layer(h, p, W1, b1, W2, b2)
    p = _segsum_sc(h, src, dst, pad_src, pad_dst, n, n_pad, ch)
    out = _mlp_layer_head(h, p, W1_2, b1_2, W2_2, b2_2, Wh, bh)
    return (out.reshape(n), None)


# speedup vs baseline: 1.1196x; 1.0211x over previous
import functools

import jax
import jax.numpy as jnp
from jax import lax
from jax.experimental import pallas as pl
from jax.experimental.pallas import tpu as pltpu
from jax.experimental.pallas import tpu_sc as plsc

NC = 2
NS = 16
NW = NC * NS
CHUNK = 80
BLK = 16
NBUF = 4


def _segsum_sc(h, src_p, dst_p, psrc, pdst, n_nodes, n_pad, ch):
    d = h.shape[1]
    per_w = src_p.shape[0] // NW
    assert per_w * NW == src_p.shape[0] and per_w % 8 == 0
    assert (per_w - (ch // BLK - 1) * BLK * CHUNK) % 8 == 0
    rows_per_sub = n_pad // NS
    out_full = ((n_nodes + NS - 1) // NS + 7) // 8 * 8
    out_last = n_nodes - out_full * (NS - 1)
    assert 0 < out_last <= out_full and out_full % 8 == 0
    mesh = plsc.VectorSubcoreMesh(
        core_axis_name="c", subcore_axis_name="s", num_cores=NC, num_subcores=NS
    )

    @functools.partial(
        pl.kernel,
        out_type=jax.ShapeDtypeStruct((NC, n_nodes, d), jnp.float32),
        mesh=mesh,
        scratch_types=(
            [pltpu.VMEM((BLK * CHUNK,), jnp.int32) for _ in range(3)]
            + [pltpu.VMEM((BLK * CHUNK,), jnp.int32) for _ in range(3)]
            + [pltpu.VMEM((CHUNK, d), jnp.float32) for _ in range(NBUF)]
            + [pltpu.VMEM_SHARED((n_pad, d), jnp.float32)]
            + [pltpu.SemaphoreType.DMA] * (4 + 2 * NBUF)
        ),
    )
    def seg_kernel(h_hbm, src_hbm, dst_hbm, psrc_hbm, pdst_hbm, out_hbm,
                   sb0, sb1, sb2, db0, db1, db2, r0, r1, r2, r3, agg_sh,
                   si0, si1, si2, zsem, g0, g1, g2, g3, s0, s1, s2, s3):
        c = lax.axis_index("c")
        s = lax.axis_index("s")
        wid = c * NS + s
        nblk = ch // BLK
        padw = ch * CHUNK - per_w
        tail_real = per_w - (nblk - 1) * BLK * CHUNK
        idx_sets = ((sb0, db0, si0), (sb1, db1, si1), (sb2, db2, si2))
        rows = (r0, r1, r2, r3)
        gsem = (g0, g1, g2, g3)
        ssem = (s0, s1, s2, s3)

        def issue_idx(b):
            sb, db, smi = idx_sets[b % 3]
            base = wid * per_w + b * BLK * CHUNK
            if b < nblk - 1:
                c0 = pltpu.async_copy(src_hbm.at[pl.ds(base, BLK * CHUNK)], sb, smi)
                c1 = pltpu.async_copy(dst_hbm.at[pl.ds(base, BLK * CHUNK)], db, smi)
                return (c0, c1)
            c0 = pltpu.async_copy(
                src_hbm.at[pl.ds(base, tail_real)], sb.at[pl.ds(0, tail_real)], smi)
            c1 = pltpu.async_copy(
                dst_hbm.at[pl.ds(base, tail_real)], db.at[pl.ds(0, tail_real)], smi)
            c2 = pltpu.async_copy(
                psrc_hbm.at[pl.ds(wid * padw, padw)], sb.at[pl.ds(tail_real, padw)], smi)
            c3 = pltpu.async_copy(
                pdst_hbm.at[pl.ds(wid * padw, padw)], db.at[pl.ds(tail_real, padw)], smi)
            return (c0, c1, c2, c3)

        pend = issue_idx(0)

        @pl.loop(0, CHUNK)
        def _zr(r):
            @pl.loop(0, d, step=16)
            def _zc(cc):
                r0[r, pl.ds(cc, 16)] = jnp.zeros((16,), jnp.float32)

        @pl.loop(0, rows_per_sub // CHUNK)
        def _zs(kz):
            pltpu.async_copy(
                r0, agg_sh.at[pl.ds(s * rows_per_sub + kz * CHUNK, CHUNK)], zsem
            )

        @pl.loop(0, rows_per_sub // CHUNK)
        def _zw(kz):
            pltpu.make_async_copy(
                r0, agg_sh.at[pl.ds(s * rows_per_sub, CHUNK)], zsem
            ).wait()

        plsc.subcore_barrier()

        def _wait(buf, sem):
            pltpu.make_async_copy(h_hbm.at[pl.ds(0, CHUNK)], buf, sem).wait()

        def dslice(buf, i):
            return buf.at[pl.ds(i * CHUNK, CHUNK)]

        for b in range(nblk):
            sb, db, _ = idx_sets[b % 3]
            dbp = idx_sets[(b - 1) % 3][1]
            for p in pend:
                p.wait()
            if b + 1 < nblk:
                pend = issue_idx(b + 1)
            if b > 0:
                _wait(rows[2], gsem[2])
                pltpu.async_copy(rows[2], agg_sh.at[dslice(dbp, BLK - 2)], s2, add=True)
                _wait(rows[3], gsem[3])
                pltpu.async_copy(rows[3], agg_sh.at[dslice(dbp, BLK - 1)], s3, add=True)

            first = b == 0

            @pl.loop(0, BLK, step=NBUF)
            def _go(j, sb=sb, db=db, first=first):
                for k in range(NBUF):
                    k2 = (k + 2) % NBUF

                    def _free(k=k):
                        _wait(rows[k], ssem[k])

                    def _scat(k2=k2, k=k, db=db):
                        _wait(rows[k2], gsem[k2])
                        pltpu.async_copy(
                            rows[k2], agg_sh.at[dslice(db, j + k - 2)], ssem[k2],
                            add=True,
                        )

                    if first:
                        pl.when(j >= NBUF)(_free)
                    else:
                        _free()
                    if k < 2:
                        pl.when(j >= NBUF)(_scat)
                    pltpu.async_copy(h_hbm.at[dslice(sb, j + k)], rows[k], gsem[k])
                    if k >= 2:
                        _scat()

        dbl = idx_sets[(nblk - 1) % 3][1]
        _wait(rows[2], gsem[2])
        pltpu.async_copy(rows[2], agg_sh.at[dslice(dbl, BLK - 2)], s2, add=True)
        _wait(rows[3], gsem[3])
        pltpu.async_copy(rows[3], agg_sh.at[dslice(dbl, BLK - 1)], s3, add=True)
        for k in range(NBUF):
            _wait(rows[k], ssem[k])

        plsc.subcore_barrier()

        @pl.when(s < NS - 1)
        def _cp_full():
            pltpu.sync_copy(
                agg_sh.at[pl.ds(s * out_full, out_full)],
                out_hbm.at[c, pl.ds(s * out_full, out_full)],
            )

        @pl.when(s == NS - 1)
        def _cp_last():
            pltpu.sync_copy(
                agg_sh.at[pl.ds((NS - 1) * out_full, out_last)],
                out_hbm.at[c, pl.ds((NS - 1) * out_full, out_last)],
            )

    return seg_kernel(h, src_p, dst_p, psrc, pdst)


def _mlp_layer(x, p, W1, b1, W2, b2, block=2000):
    n, d = x.shape

    def body(x_r, p_r, w1_r, b1_r, w2_r, b2_r, o_r):
        z = x_r[...] + p_r[0] + p_r[1]
        h1 = jnp.maximum(
            jnp.dot(z, w1_r[...], preferred_element_type=jnp.float32) + b1_r[...], 0.0
        )
        h2 = jnp.dot(h1, w2_r[...], preferred_element_type=jnp.float32) + b2_r[...]
        o_r[...] = jnp.maximum(h2, 0.0)

    return pl.pallas_call(
        body,
        grid=(n // block,),
        in_specs=[
            pl.BlockSpec((block, d), lambda i: (i, 0)),
            pl.BlockSpec((NC, block, d), lambda i: (0, i, 0)),
            pl.BlockSpec((d, d), lambda i: (0, 0)),
            pl.BlockSpec((1, d), lambda i: (0, 0)),
            pl.BlockSpec((d, d), lambda i: (0, 0)),
            pl.BlockSpec((1, d), lambda i: (0, 0)),
        ],
        out_specs=pl.BlockSpec((block, d), lambda i: (i, 0)),
        out_shape=jax.ShapeDtypeStruct((n, d), jnp.float32),
    )(x, p, W1, b1.reshape(1, d), W2, b2.reshape(1, d))


def _mlp_layer_head(x, p, W1, b1, W2, b2, Wh, bh, block=2000):
    n, d = x.shape

    def body(x_r, p_r, w1_r, b1_r, w2_r, b2_r, wh_r, bh_r, o_r):
        z = x_r[...] + p_r[0] + p_r[1]
        h1 = jnp.maximum(
            jnp.dot(z, w1_r[...], preferred_element_type=jnp.float32) + b1_r[...], 0.0
        )
        h2 = jnp.dot(h1, w2_r[...], preferred_element_type=jnp.float32) + b2_r[...]
        h2 = jnp.maximum(h2, 0.0)
        o_r[...] = jnp.dot(h2, wh_r[...], preferred_element_type=jnp.float32) + bh_r[...]

    return pl.pallas_call(
        body,
        grid=(n // block,),
        in_specs=[
            pl.BlockSpec((block, d), lambda i: (i, 0)),
            pl.BlockSpec((NC, block, d), lambda i: (0, i, 0)),
            pl.BlockSpec((d, d), lambda i: (0, 0)),
            pl.BlockSpec((1, d), lambda i: (0, 0)),
            pl.BlockSpec((d, d), lambda i: (0, 0)),
            pl.BlockSpec((1, d), lambda i: (0, 0)),
            pl.BlockSpec((d, 1), lambda i: (0, 0)),
            pl.BlockSpec((1, 1), lambda i: (0, 0)),
        ],
        out_specs=pl.BlockSpec((block, 1), lambda i: (i, 0)),
        out_shape=jax.ShapeDtypeStruct((n, 1), jnp.float32),
    )(x, p, W1, b1.reshape(1, d), W2, b2.reshape(1, d), Wh, bh.reshape(1, 1))


def kernel(x, edge_index, W1_0, b1_0, W2_0, b2_0, W1_1, b1_1, W2_1, b2_1,
           W1_2, b1_2, W2_2, b2_2, Wh, bh):
    n, d = x.shape
    e = edge_index.shape[1]

    ch = (e + NW * CHUNK - 1) // (NW * CHUNK)
    ch = (ch + BLK - 1) // BLK * BLK
    n_pad = (n + 1 + NS * CHUNK - 1) // (NS * CHUNK) * (NS * CHUNK)

    src = edge_index[0].reshape(e).astype(jnp.int32)
    dst = edge_index[1].reshape(e).astype(jnp.int32)
    per_w = e // NW
    padw = ch * CHUNK - per_w
    wids = jnp.arange(NW, dtype=jnp.int32)[:, None]
    lanes = jnp.arange(padw, dtype=jnp.int32)[None, :]
    pad_src = ((wids * padw + lanes) % n).reshape(NW * padw)
    pad_dst = (n + (wids * 7 + lanes) % (n_pad - n)).reshape(NW * padw)

    h = x
    layers = [(W1_0, b1_0, W2_0, b2_0), (W1_1, b1_1, W2_1, b2_1)]
    for (W1, b1, W2, b2) in layers:
        p = _segsum_sc(h, src, dst, pad_src, pad_dst, n, n_pad, ch)
        h = _mlp_layer(h, p, W1, b1, W2, b2)
    p = _segsum_sc(h, src, dst, pad_src, pad_dst, n, n_pad, ch)
    out = _mlp_layer_head(h, p, W1_2, b1_2, W2_2, b2_2, Wh, bh)
    return (out.reshape(n), None)
